# baseline (speedup 1.0000x reference)
import functools

import jax
import jax.numpy as jnp
import numpy as np
from jax.experimental import pallas as pl
from jax.experimental.pallas import tpu as pltpu

_ALPHA = 3
_ALIVE_THRESHOLD = 0.1
_G = 8
_NSUB = 4


def _nca_kernel(x_ref, gxy_ref, w1bd_ref, b1_ref, w2bd_ref,
                selx_ref, selb_ref, mask_ref, o_ref, *, height, width,
                n_channels):
    H, W = height, width
    C = n_channels
    HW = H * W
    rows = x_ref.shape[0]
    x_all = x_ref[...].reshape(rows, HW)
    gsz = _G * C
    for s in range(_NSUB):
        _nca_group(x_all[s * gsz:(s + 1) * gsz], gxy_ref, w1bd_ref,
                   b1_ref, w2bd_ref, selx_ref, selb_ref, mask_ref,
                   o_ref.at[s * _G:(s + 1) * _G], H, W)


def _nca_group(x, gxy_ref, w1bd_ref, b1_ref, w2bd_ref,
               selx_ref, selb_ref, mask_ref, o_sub, H, W):
    HW = H * W
    xb = x.astype(jnp.bfloat16)

    grads = jnp.dot(xb, gxy_ref[...],
                    preferred_element_type=jnp.float32).astype(jnp.bfloat16)
    grad_x = grads[:, :HW]
    grad_y = grads[:, HW:]

    percept = jnp.concatenate([xb, grad_x, grad_y], axis=0)
    h = jnp.dot(w1bd_ref[...], percept,
                preferred_element_type=jnp.float32) + b1_ref[...]
    h = jnp.maximum(h, 0.0).astype(jnp.bfloat16)
    ds = jnp.dot(w2bd_ref[...], h,
                 preferred_element_type=jnp.float32)

    raw = x + ds * mask_ref[...]

    alpha = jnp.dot(selx_ref[...], raw.astype(jnp.bfloat16),
                    preferred_element_type=jnp.float32)
    lane = jax.lax.broadcasted_iota(jnp.int32, alpha.shape, 1)
    wcol = lane % W
    hrow = lane // W
    neg_inf = jnp.float32(-jnp.inf)
    left = jnp.where(wcol >= 1, pltpu.roll(alpha, 1, axis=1), neg_inf)
    right = jnp.where(wcol <= W - 2, pltpu.roll(alpha, HW - 1, axis=1), neg_inf)
    pw = jnp.maximum(alpha, jnp.maximum(left, right))
    up = jnp.where(hrow >= 1, pltpu.roll(pw, W, axis=1), neg_inf)
    down = jnp.where(hrow <= H - 2, pltpu.roll(pw, HW - W, axis=1), neg_inf)
    pooled = jnp.maximum(pw, jnp.maximum(up, down))
    alive = (pooled > _ALIVE_THRESHOLD).astype(jnp.bfloat16)
    alive_b = jnp.dot(selb_ref[...], alive,
                      preferred_element_type=jnp.float32)

    o_sub[...] = (raw * alive_b).reshape(o_sub.shape)


def _sobel_ops(H, W):
    HW = H * W
    idx = np.arange(HW)
    h, w = idx // W, idx % W
    gx = np.zeros((HW, HW), np.float32)
    gy = np.zeros((HW, HW), np.float32)
    for d, a in ((-1, 1.0), (0, 2.0), (1, 1.0)):
        for s, sign in ((1, 1.0), (-1, -1.0)):
            src = ((h + d) % H) * W + (w + s) % W
            np.add.at(gx, (src, idx), sign * a)
            src = ((h - s) % H) * W + (w + d) % W
            np.add.at(gy, (src, idx), sign * a)
    return gx, gy


def kernel(x_nchw, w1, b1, w2, rand_mask):
    B, C, H, W = x_nchw.shape
    HW = H * W

    gx_np, gy_np = _sobel_ops(H, W)
    gxy = jnp.asarray(np.concatenate([gx_np, gy_np], axis=1), jnp.bfloat16)

    eye_g = jnp.eye(_G, dtype=jnp.float32)
    w1bd = jnp.concatenate(
        [jnp.kron(eye_g, w1[t * C:(t + 1) * C].T) for t in range(3)],
        axis=1).astype(jnp.bfloat16)
    w2bd = jnp.kron(eye_g, w2.T).astype(jnp.bfloat16)
    b1col = jnp.tile(b1, _G)[:, None]

    selx_np = np.zeros((_G, _G * C), np.float32)
    selx_np[np.arange(_G), np.arange(_G) * C + _ALPHA] = 1.0
    selb_np = np.zeros((_G * C, _G), np.float32)
    selb_np[np.arange(_G * C), np.arange(_G * C) // C] = 1.0
    selx = jnp.asarray(selx_np, jnp.bfloat16)
    selb = jnp.asarray(selb_np, jnp.bfloat16)
    mask_flat = rand_mask.reshape(1, HW)

    x3 = x_nchw.reshape(B * C, H, W)
    rows = _NSUB * _G * C
    n_blocks = B * C // rows
    ncores = 2
    per_core = n_blocks // ncores
    body = functools.partial(_nca_kernel, height=H, width=W, n_channels=C)
    out = pl.pallas_call(
        body,
        grid=(ncores, per_core),
        out_shape=jax.ShapeDtypeStruct((B, C, H, W), jnp.float32),
        in_specs=[
            pl.BlockSpec((rows, H, W),
                         lambda c, i: (c * per_core + i, 0, 0)),
            pl.BlockSpec(gxy.shape, lambda c, i: (0, 0)),
            pl.BlockSpec(w1bd.shape, lambda c, i: (0, 0)),
            pl.BlockSpec(b1col.shape, lambda c, i: (0, 0)),
            pl.BlockSpec(w2bd.shape, lambda c, i: (0, 0)),
            pl.BlockSpec(selx.shape, lambda c, i: (0, 0)),
            pl.BlockSpec(selb.shape, lambda c, i: (0, 0)),
            pl.BlockSpec(mask_flat.shape, lambda c, i: (0, 0)),
        ],
        out_specs=pl.BlockSpec((_NSUB * _G, C, H, W),
                               lambda c, i: (c * per_core + i, 0, 0, 0)),
        compiler_params=pltpu.CompilerParams(
            dimension_semantics=("parallel", "arbitrary")),
    )(x3, gxy, w1bd, b1col, w2bd, selx, selb, mask_flat)
    return out


# speedup vs baseline: 1.2607x; 1.2607x over previous
import functools

import jax
import jax.numpy as jnp
import numpy as np
from jax.experimental import pallas as pl
from jax.experimental.pallas import tpu as pltpu

_ALPHA = 3
_ALIVE_THRESHOLD = 0.1
_G = 8
_NSUB = 4


def _nca_kernel(x_ref, gxy_ref, w1bd_ref, b1_ref, w2bd_ref,
                selx_ref, selb_ref, mask_ref, o_ref, *, height, width,
                n_channels):
    H, W = height, width
    C = n_channels
    HW = H * W
    rows = x_ref.shape[0]
    gsz = _G * C
    x_all = x_ref[...].reshape(rows, HW)
    xb_all = x_all.astype(jnp.bfloat16)

    grads = jnp.dot(xb_all, gxy_ref[...],
                    preferred_element_type=jnp.float32).astype(jnp.bfloat16)

    raws = []
    for s in range(_NSUB):
        sl = slice(s * gsz, (s + 1) * gsz)
        percept = jnp.concatenate(
            [xb_all[sl], grads[sl, :HW], grads[sl, HW:]], axis=0)
        h = jnp.dot(w1bd_ref[...], percept,
                    preferred_element_type=jnp.float32) + b1_ref[...]
        h = jnp.maximum(h, 0.0).astype(jnp.bfloat16)
        ds = jnp.dot(w2bd_ref[...], h,
                     preferred_element_type=jnp.float32)
        raws.append(x_all[sl] + ds * mask_ref[...])
    raw = jnp.concatenate(raws, axis=0)

    alpha = jnp.dot(selx_ref[...], raw.astype(jnp.bfloat16),
                    preferred_element_type=jnp.float32)
    lane = jax.lax.broadcasted_iota(jnp.int32, alpha.shape, 1)
    wcol = lane % W
    hrow = lane // W
    neg_inf = jnp.float32(-jnp.inf)
    left = jnp.where(wcol >= 1, pltpu.roll(alpha, 1, axis=1), neg_inf)
    right = jnp.where(wcol <= W - 2, pltpu.roll(alpha, HW - 1, axis=1), neg_inf)
    pw = jnp.maximum(alpha, jnp.maximum(left, right))
    up = jnp.where(hrow >= 1, pltpu.roll(pw, W, axis=1), neg_inf)
    down = jnp.where(hrow <= H - 2, pltpu.roll(pw, HW - W, axis=1), neg_inf)
    pooled = jnp.maximum(pw, jnp.maximum(up, down))
    alive = (pooled > _ALIVE_THRESHOLD).astype(jnp.bfloat16)
    alive_b = jnp.dot(selb_ref[...], alive,
                      preferred_element_type=jnp.float32)

    o_ref[...] = (raw * alive_b).reshape(o_ref.shape)


def _sobel_ops(H, W):
    HW = H * W
    idx = np.arange(HW)
    h, w = idx // W, idx % W
    gx = np.zeros((HW, HW), np.float32)
    gy = np.zeros((HW, HW), np.float32)
    for d, a in ((-1, 1.0), (0, 2.0), (1, 1.0)):
        for s, sign in ((1, 1.0), (-1, -1.0)):
            src = ((h + d) % H) * W + (w + s) % W
            np.add.at(gx, (src, idx), sign * a)
            src = ((h - s) % H) * W + (w + d) % W
            np.add.at(gy, (src, idx), sign * a)
    return gx, gy


def kernel(x_nchw, w1, b1, w2, rand_mask):
    B, C, H, W = x_nchw.shape
    HW = H * W

    gx_np, gy_np = _sobel_ops(H, W)
    gxy = jnp.asarray(np.concatenate([gx_np, gy_np], axis=1), jnp.bfloat16)

    eye_g = jnp.eye(_G, dtype=jnp.float32)
    w1bd = jnp.concatenate(
        [jnp.kron(eye_g, w1[t * C:(t + 1) * C].T) for t in range(3)],
        axis=1).astype(jnp.bfloat16)
    w2bd = jnp.kron(eye_g, w2.T).astype(jnp.bfloat16)
    b1col = jnp.tile(b1, _G)[:, None]

    E = _NSUB * _G
    selx_np = np.zeros((E, E * C), np.float32)
    selx_np[np.arange(E), np.arange(E) * C + _ALPHA] = 1.0
    selb_np = np.zeros((E * C, E), np.float32)
    selb_np[np.arange(E * C), np.arange(E * C) // C] = 1.0
    selx = jnp.asarray(selx_np, jnp.bfloat16)
    selb = jnp.asarray(selb_np, jnp.bfloat16)
    mask_flat = rand_mask.reshape(1, HW)

    x3 = x_nchw.reshape(B * C, H, W)
    rows = _NSUB * _G * C
    n_blocks = B * C // rows
    ncores = 2
    per_core = n_blocks // ncores
    body = functools.partial(_nca_kernel, height=H, width=W, n_channels=C)
    out = pl.pallas_call(
        body,
        grid=(ncores, per_core),
        out_shape=jax.ShapeDtypeStruct((B, C, H, W), jnp.float32),
        in_specs=[
            pl.BlockSpec((rows, H, W),
                         lambda c, i: (c * per_core + i, 0, 0)),
            pl.BlockSpec(gxy.shape, lambda c, i: (0, 0)),
            pl.BlockSpec(w1bd.shape, lambda c, i: (0, 0)),
            pl.BlockSpec(b1col.shape, lambda c, i: (0, 0)),
            pl.BlockSpec(w2bd.shape, lambda c, i: (0, 0)),
            pl.BlockSpec(selx.shape, lambda c, i: (0, 0)),
            pl.BlockSpec(selb.shape, lambda c, i: (0, 0)),
            pl.BlockSpec(mask_flat.shape, lambda c, i: (0, 0)),
        ],
        out_specs=pl.BlockSpec((_NSUB * _G, C, H, W),
                               lambda c, i: (c * per_core + i, 0, 0, 0)),
        compiler_params=pltpu.CompilerParams(
            dimension_semantics=("parallel", "arbitrary")),
    )(x3, gxy, w1bd, b1col, w2bd, selx, selb, mask_flat)
    return out


# speedup vs baseline: 1.3061x; 1.0360x over previous
import functools

import jax
import jax.numpy as jnp
import numpy as np
from jax.experimental import pallas as pl
from jax.experimental.pallas import tpu as pltpu

_ALPHA = 3
_ALIVE_THRESHOLD = 0.1
_G = 8
_NSUB = 8


def _nca_kernel(x_ref, gxy_ref, w1bd_ref, b1_ref, w2bd_ref,
                selx_ref, selb_ref, mask_ref, o_ref, *, height, width,
                n_channels):
    H, W = height, width
    C = n_channels
    HW = H * W
    rows = x_ref.shape[0]
    gsz = _G * C
    x_all = x_ref[...].reshape(rows, HW)
    xb_all = x_all.astype(jnp.bfloat16)

    grads = jnp.dot(xb_all, gxy_ref[...],
                    preferred_element_type=jnp.float32).astype(jnp.bfloat16)

    raws = []
    for s in range(_NSUB):
        sl = slice(s * gsz, (s + 1) * gsz)
        percept = jnp.concatenate(
            [xb_all[sl], grads[sl, :HW], grads[sl, HW:]], axis=0)
        h = jnp.dot(w1bd_ref[...], percept,
                    preferred_element_type=jnp.float32) + b1_ref[...]
        h = jnp.maximum(h, 0.0).astype(jnp.bfloat16)
        ds = jnp.dot(w2bd_ref[...], h,
                     preferred_element_type=jnp.float32)
        raws.append(x_all[sl] + ds * mask_ref[...])
    raw = jnp.concatenate(raws, axis=0)

    alpha = jnp.dot(selx_ref[...], raw.astype(jnp.bfloat16),
                    preferred_element_type=jnp.float32)
    lane = jax.lax.broadcasted_iota(jnp.int32, alpha.shape, 1)
    wcol = lane % W
    hrow = lane // W
    neg_inf = jnp.float32(-jnp.inf)
    left = jnp.where(wcol >= 1, pltpu.roll(alpha, 1, axis=1), neg_inf)
    right = jnp.where(wcol <= W - 2, pltpu.roll(alpha, HW - 1, axis=1), neg_inf)
    pw = jnp.maximum(alpha, jnp.maximum(left, right))
    up = jnp.where(hrow >= 1, pltpu.roll(pw, W, axis=1), neg_inf)
    down = jnp.where(hrow <= H - 2, pltpu.roll(pw, HW - W, axis=1), neg_inf)
    pooled = jnp.maximum(pw, jnp.maximum(up, down))
    alive = (pooled > _ALIVE_THRESHOLD).astype(jnp.bfloat16)
    alive_b = jnp.dot(selb_ref[...], alive,
                      preferred_element_type=jnp.float32)

    o_ref[...] = (raw * alive_b).reshape(o_ref.shape)


def _sobel_ops(H, W):
    HW = H * W
    idx = np.arange(HW)
    h, w = idx // W, idx % W
    gx = np.zeros((HW, HW), np.float32)
    gy = np.zeros((HW, HW), np.float32)
    for d, a in ((-1, 1.0), (0, 2.0), (1, 1.0)):
        for s, sign in ((1, 1.0), (-1, -1.0)):
            src = ((h + d) % H) * W + (w + s) % W
            np.add.at(gx, (src, idx), sign * a)
            src = ((h - s) % H) * W + (w + d) % W
            np.add.at(gy, (src, idx), sign * a)
    return gx, gy


def kernel(x_nchw, w1, b1, w2, rand_mask):
    B, C, H, W = x_nchw.shape
    HW = H * W

    gx_np, gy_np = _sobel_ops(H, W)
    gxy = jnp.asarray(np.concatenate([gx_np, gy_np], axis=1), jnp.bfloat16)

    eye_g = jnp.eye(_G, dtype=jnp.float32)
    w1bd = jnp.concatenate(
        [jnp.kron(eye_g, w1[t * C:(t + 1) * C].T) for t in range(3)],
        axis=1).astype(jnp.bfloat16)
    w2bd = jnp.kron(eye_g, w2.T).astype(jnp.bfloat16)
    b1col = jnp.tile(b1, _G)[:, None]

    E = _NSUB * _G
    selx_np = np.zeros((E, E * C), np.float32)
    selx_np[np.arange(E), np.arange(E) * C + _ALPHA] = 1.0
    selb_np = np.zeros((E * C, E), np.float32)
    selb_np[np.arange(E * C), np.arange(E * C) // C] = 1.0
    selx = jnp.asarray(selx_np, jnp.bfloat16)
    selb = jnp.asarray(selb_np, jnp.bfloat16)
    mask_flat = rand_mask.reshape(1, HW)

    x3 = x_nchw.reshape(B * C, H, W)
    rows = _NSUB * _G * C
    n_blocks = B * C // rows
    ncores = 2
    per_core = n_blocks // ncores
    body = functools.partial(_nca_kernel, height=H, width=W, n_channels=C)
    out = pl.pallas_call(
        body,
        grid=(ncores, per_core),
        out_shape=jax.ShapeDtypeStruct((B, C, H, W), jnp.float32),
        in_specs=[
            pl.BlockSpec((rows, H, W),
                         lambda c, i: (c * per_core + i, 0, 0)),
            pl.BlockSpec(gxy.shape, lambda c, i: (0, 0)),
            pl.BlockSpec(w1bd.shape, lambda c, i: (0, 0)),
            pl.BlockSpec(b1col.shape, lambda c, i: (0, 0)),
            pl.BlockSpec(w2bd.shape, lambda c, i: (0, 0)),
            pl.BlockSpec(selx.shape, lambda c, i: (0, 0)),
            pl.BlockSpec(selb.shape, lambda c, i: (0, 0)),
            pl.BlockSpec(mask_flat.shape, lambda c, i: (0, 0)),
        ],
        out_specs=pl.BlockSpec((_NSUB * _G, C, H, W),
                               lambda c, i: (c * per_core + i, 0, 0, 0)),
        compiler_params=pltpu.CompilerParams(
            dimension_semantics=("parallel", "arbitrary")),
    )(x3, gxy, w1bd, b1col, w2bd, selx, selb, mask_flat)
    return out
